# baseline (speedup 1.0000x reference)
import functools

import numpy as np
import jax
import jax.numpy as jnp
from jax import lax
from jax.experimental import pallas as pl
from jax.experimental.pallas import tpu as pltpu
from jax.experimental.pallas import tpu_sc as plsc

_N, _H, _W, _C = 4, 384, 384, 96
_OH = _OW = 224
_HALF = 112
_SEGW = 192
_NTASK = _N * _OH * 2
_NWORK = 32
_TPW = _NTASK // _NWORK
_ROWLEN = _SEGW * _C
_OUTLEN = _HALF * _C


def _axis_tables():
    scale = np.float32(_W / _OW)
    i = np.arange(_OW, dtype=np.float32)
    src = (i + np.float32(0.5)) * scale - np.float32(0.5)
    src = np.maximum(src, np.float32(0.0))
    lo = np.floor(src).astype(np.int32)
    frac = (src - lo.astype(np.float32)).astype(np.float32)
    hi = lo + (lo < _W - 1).astype(np.int32)
    return lo, hi, frac


def _tables():
    lo, hi, frac = _axis_tables()
    lane = np.arange(16, dtype=np.int32)

    h_of_j = np.arange(_OW, dtype=np.int32) // _HALF
    xa = (lo - _SEGW * h_of_j) * _C
    xb = (hi - _SEGW * h_of_j) * _C
    xa_t = (xa[:, None] + lane[None, :]).reshape(-1)
    xb_t = (xb[:, None] + lane[None, :]).reshape(-1)
    fx_t = np.broadcast_to(frac[:, None], (_OW, 16)).astype(np.float32).reshape(-1).copy()

    t = np.arange(_NTASK, dtype=np.int32)
    h = t % 2
    i_out = (t // 2) % _OH
    n = t // (2 * _OH)
    s0 = (n * _H + lo[i_out]) * 2 + h
    s1 = (n * _H + hi[i_out]) * 2 + h
    idx_t = np.zeros((_NTASK, 8), dtype=np.int32)
    idx_t[:, 0] = s0
    idx_t[:, 1] = s1
    fy_t = np.broadcast_to(frac[i_out][:, None], (_NTASK, 16)).astype(np.float32).reshape(-1).copy()
    return xa_t, xb_t, fx_t, fy_t, idx_t


_XA, _XB, _FX, _FY, _IDX = _tables()
_ROW0 = np.zeros(16, dtype=np.int32)
_ROW1 = np.ones(16, dtype=np.int32)


def _sc_resize(img3):
    mesh = plsc.VectorSubcoreMesh(core_axis_name="c", subcore_axis_name="s")

    @functools.partial(
        pl.kernel,
        out_type=jax.ShapeDtypeStruct((_NTASK, _OUTLEN), jnp.float32),
        mesh=mesh,
        scratch_types=[
            pltpu.VMEM((_OW * 16,), jnp.int32),
            pltpu.VMEM((_OW * 16,), jnp.int32),
            pltpu.VMEM((_OW * 16,), jnp.float32),
            pltpu.VMEM((_TPW * 16,), jnp.float32),
            pltpu.VMEM((2 * _ROWLEN,), jnp.float32),
            pltpu.VMEM((_OUTLEN,), jnp.float32),
            pltpu.SemaphoreType.DMA,
        ],
        compiler_params=pltpu.CompilerParams(needs_layout_passes=False),
    )
    def run(img_hbm, xa_hbm, xb_hbm, fx_hbm, fy_hbm, out_hbm,
            xa_v, xb_v, fx_v, fy_v, rows_v, out_v, sem):
        wid = lax.axis_index("s") * 2 + lax.axis_index("c")
        base = wid * _TPW
        pltpu.sync_copy(xa_hbm, xa_v)
        pltpu.sync_copy(xb_hbm, xb_v)
        pltpu.sync_copy(fx_hbm, fx_v)
        pltpu.sync_copy(fy_hbm.at[pl.ds(base * 16, _TPW * 16)], fy_v)

        def task(k, carry):
            t = base + k
            h = lax.rem(t, 2)
            i = lax.rem(lax.div(t, 2), _OH)
            n = lax.div(t, 2 * _OH)
            y0 = lax.div(24 * i + 5, 14)
            s0 = (n * _H + y0) * 2 + h
            cp0 = pltpu.async_copy(
                img_hbm.at[s0], rows_v.at[pl.ds(0, _ROWLEN)], sem)
            cp1 = pltpu.async_copy(
                img_hbm.at[s0 + 2], rows_v.at[pl.ds(_ROWLEN, _ROWLEN)], sem)
            cp0.wait()
            cp1.wait()
            fyv = fy_v[pl.ds(k * 16, 16)]
            jbase = h * _HALF

            def jloop(j, c2):
                jj = (jbase + j) * 16
                iav = xa_v[pl.ds(jj, 16)]
                ibv = xb_v[pl.ds(jj, 16)]
                fxv = fx_v[pl.ds(jj, 16)]
                obase = j * _C
                for c in range(0, _C, 16):
                    ia = iav + c
                    ib = ibv + c
                    a0 = plsc.load_gather(rows_v, [ia])
                    b0 = plsc.load_gather(rows_v, [ib])
                    a1 = plsc.load_gather(rows_v, [ia + _ROWLEN])
                    b1 = plsc.load_gather(rows_v, [ib + _ROWLEN])
                    t0 = a0 + fxv * (b0 - a0)
                    t1 = a1 + fxv * (b1 - a1)
                    out_v[pl.ds(obase + c, 16)] = t0 + fyv * (t1 - t0)
                return c2

            lax.fori_loop(0, _HALF, jloop, 0)
            pltpu.sync_copy(out_v, out_hbm.at[base + k])
            return carry

        lax.fori_loop(0, _TPW, task, 0)

    return run(img3, _XA, _XB, _FX, _FY)


def kernel(img):
    img3 = img.reshape(_N * _H * 2, _ROWLEN)
    out = _sc_resize(img3)
    return out.reshape(_N, _OH, _OW, _C)


# speedup vs baseline: 1.0921x; 1.0921x over previous
import functools

import numpy as np
import jax
import jax.numpy as jnp
from jax import lax
from jax.experimental import pallas as pl
from jax.experimental.pallas import tpu as pltpu
from jax.experimental.pallas import tpu_sc as plsc

_N, _H, _W, _C = 4, 384, 384, 96
_OH = _OW = 224
_HALF = 112
_SEGW = 192
_NTASK = _N * _OH * 2
_NWORK = 32
_TPW = _NTASK // _NWORK
_ROWLEN = _SEGW * _C
_OUTLEN = _HALF * _C


def _axis_tables():
    scale = np.float32(_W / _OW)
    i = np.arange(_OW, dtype=np.float32)
    src = (i + np.float32(0.5)) * scale - np.float32(0.5)
    src = np.maximum(src, np.float32(0.0))
    lo = np.floor(src).astype(np.int32)
    frac = (src - lo.astype(np.float32)).astype(np.float32)
    hi = lo + (lo < _W - 1).astype(np.int32)
    return lo, hi, frac


def _tables():
    lo, hi, frac = _axis_tables()
    lane = np.arange(16, dtype=np.int32)

    h_of_j = np.arange(_OW, dtype=np.int32) // _HALF
    xa = (lo - _SEGW * h_of_j) * _C
    xb = (hi - _SEGW * h_of_j) * _C
    xa_t = (xa[:, None] + lane[None, :]).reshape(-1)
    xb_t = (xb[:, None] + lane[None, :]).reshape(-1)
    fx_t = np.broadcast_to(frac[:, None], (_OW, 16)).astype(np.float32).reshape(-1).copy()

    t = np.arange(_NTASK, dtype=np.int32)
    h = t % 2
    i_out = (t // 2) % _OH
    n = t // (2 * _OH)
    s0 = (n * _H + lo[i_out]) * 2 + h
    s1 = (n * _H + hi[i_out]) * 2 + h
    idx_t = np.zeros((_NTASK, 8), dtype=np.int32)
    idx_t[:, 0] = s0
    idx_t[:, 1] = s1
    fy_t = np.broadcast_to(frac[i_out][:, None], (_NTASK, 16)).astype(np.float32).reshape(-1).copy()
    return xa_t, xb_t, fx_t, fy_t, idx_t


_XA, _XB, _FX, _FY, _IDX = _tables()
_ROW0 = np.zeros(16, dtype=np.int32)
_ROW1 = np.ones(16, dtype=np.int32)


def _sc_resize(img3):
    mesh = plsc.VectorSubcoreMesh(core_axis_name="c", subcore_axis_name="s")

    @functools.partial(
        pl.kernel,
        out_type=jax.ShapeDtypeStruct((_NTASK, _OUTLEN), jnp.float32),
        mesh=mesh,
        scratch_types=[
            pltpu.VMEM((_OW * 16,), jnp.int32),
            pltpu.VMEM((_OW * 16,), jnp.int32),
            pltpu.VMEM((_OW * 16,), jnp.float32),
            pltpu.VMEM((_TPW * 16,), jnp.float32),
            pltpu.VMEM((4 * _ROWLEN,), jnp.float32),
            pltpu.VMEM((_OUTLEN,), jnp.float32),
            pltpu.SemaphoreType.DMA,
            pltpu.SemaphoreType.DMA,
        ],
        compiler_params=pltpu.CompilerParams(needs_layout_passes=False),
    )
    def run(img_hbm, xa_hbm, xb_hbm, fx_hbm, fy_hbm, out_hbm,
            xa_v, xb_v, fx_v, fy_v, rows_v, out_v, rsem0, rsem1):
        wid = lax.axis_index("s") * 2 + lax.axis_index("c")
        base = wid * _TPW
        pltpu.sync_copy(xa_hbm, xa_v)
        pltpu.sync_copy(xb_hbm, xb_v)
        pltpu.sync_copy(fx_hbm, fx_v)
        pltpu.sync_copy(fy_hbm.at[pl.ds(base * 16, _TPW * 16)], fy_v)

        rsems = (rsem0, rsem1)

        def row_copies(k, slot):
            t = base + k
            h = lax.rem(t, 2)
            i = lax.rem(lax.div(t, 2), _OH)
            n = lax.div(t, 2 * _OH)
            y0 = lax.div(24 * i + 5, 14)
            s0 = (n * _H + y0) * 2 + h
            sem = rsems[slot]
            off = slot * 2 * _ROWLEN
            return (
                pltpu.make_async_copy(
                    img_hbm.at[s0], rows_v.at[pl.ds(off, _ROWLEN)], sem),
                pltpu.make_async_copy(
                    img_hbm.at[s0 + 2],
                    rows_v.at[pl.ds(off + _ROWLEN, _ROWLEN)], sem),
            )

        def start_rows(k, slot):
            c0, c1 = row_copies(k, slot)
            c0.start()
            c1.start()

        def wait_rows(k, slot):
            c0, c1 = row_copies(k, slot)
            c0.wait()
            c1.wait()

        start_rows(0, 0)

        def task(k, slot):
            @pl.when(k + 1 < _TPW)
            def _():
                start_rows(k + 1, slot ^ 1)
            wait_rows(k, slot)
            soff = slot * 2 * _ROWLEN
            fyv = fy_v[pl.ds(k * 16, 16)]
            jbase = lax.rem(base + k, 2) * _HALF

            def jloop(j, c2):
                jj = (jbase + j) * 16
                iav = xa_v[pl.ds(jj, 16)]
                ibv = xb_v[pl.ds(jj, 16)]
                fxv = fx_v[pl.ds(jj, 16)]
                obase = j * _C
                for c in range(0, _C, 16):
                    ia = iav + (soff + c)
                    ib = ibv + (soff + c)
                    a0 = plsc.load_gather(rows_v, [ia])
                    b0 = plsc.load_gather(rows_v, [ib])
                    a1 = plsc.load_gather(rows_v, [ia + _ROWLEN])
                    b1 = plsc.load_gather(rows_v, [ib + _ROWLEN])
                    t0 = a0 + fxv * (b0 - a0)
                    t1 = a1 + fxv * (b1 - a1)
                    out_v[pl.ds(obase + c, 16)] = t0 + fyv * (t1 - t0)
                return c2

            lax.fori_loop(0, _HALF, jloop, 0)
            pltpu.sync_copy(out_v, out_hbm.at[base + k])

        def pair(k2, carry):
            task(2 * k2, 0)
            task(2 * k2 + 1, 1)
            return carry

        lax.fori_loop(0, _TPW // 2, pair, 0)

    return run(img3, _XA, _XB, _FX, _FY)


def kernel(img):
    img3 = img.reshape(_N * _H * 2, _ROWLEN)
    out = _sc_resize(img3)
    return out.reshape(_N, _OH, _OW, _C)


# speedup vs baseline: 1.5262x; 1.3974x over previous
import functools

import numpy as np
import jax
import jax.numpy as jnp
from jax import lax
from jax.experimental import pallas as pl
from jax.experimental.pallas import tpu as pltpu
from jax.experimental.pallas import tpu_sc as plsc

_N, _H, _W, _C = 4, 384, 384, 96
_OH = _OW = 224
_HALF = 112
_SEGW = 192
_NTASK = _N * _OH * 2
_NWORK = 32
_TPW = _NTASK // _NWORK
_ROWLEN = _SEGW * _C
_OUTLEN = _HALF * _C


def _axis_tables():
    scale = np.float32(_W / _OW)
    i = np.arange(_OW, dtype=np.float32)
    src = (i + np.float32(0.5)) * scale - np.float32(0.5)
    src = np.maximum(src, np.float32(0.0))
    lo = np.floor(src).astype(np.int32)
    frac = (src - lo.astype(np.float32)).astype(np.float32)
    hi = lo + (lo < _W - 1).astype(np.int32)
    return lo, hi, frac


def _tables():
    lo, hi, frac = _axis_tables()
    lane = np.arange(16, dtype=np.int32)

    h_of_j = np.arange(_OW, dtype=np.int32) // _HALF
    xa = (lo - _SEGW * h_of_j) * _C
    xa_t = (xa[:, None] + lane[None, :]).reshape(-1)
    fx_t = np.broadcast_to(frac[:, None], (_OW, 16)).astype(np.float32).reshape(-1).copy()

    t = np.arange(_NTASK, dtype=np.int32)
    h = t % 2
    i_out = (t // 2) % _OH
    n = t // (2 * _OH)
    s0 = (n * _H + lo[i_out]) * 2 + h
    s1 = (n * _H + hi[i_out]) * 2 + h
    fy_t = np.broadcast_to(frac[i_out][:, None], (_NTASK, 16)).astype(np.float32).reshape(-1).copy()
    return xa_t, fx_t, fy_t


_XA, _FX, _FY = _tables()
_ROW0 = np.zeros(16, dtype=np.int32)
_ROW1 = np.ones(16, dtype=np.int32)


def _sc_resize(img3):
    mesh = plsc.VectorSubcoreMesh(core_axis_name="c", subcore_axis_name="s")

    @functools.partial(
        pl.kernel,
        out_type=jax.ShapeDtypeStruct((_NTASK, _OUTLEN), jnp.float32),
        mesh=mesh,
        scratch_types=[
            pltpu.VMEM((_OW * 16,), jnp.int32),
            pltpu.VMEM((_OW * 16,), jnp.float32),
            pltpu.VMEM((_TPW * 16,), jnp.float32),
            pltpu.VMEM((4 * _ROWLEN,), jnp.float32),
            pltpu.VMEM((_OUTLEN,), jnp.float32),
            pltpu.SemaphoreType.DMA,
            pltpu.SemaphoreType.DMA,
        ],
        compiler_params=pltpu.CompilerParams(needs_layout_passes=False),
    )
    def run(img_hbm, xa_hbm, fx_hbm, fy_hbm, out_hbm,
            xa_v, fx_v, fy_v, rows_v, out_v, rsem0, rsem1):
        wid = lax.axis_index("s") * 2 + lax.axis_index("c")
        base = wid * _TPW
        pltpu.sync_copy(xa_hbm, xa_v)
        pltpu.sync_copy(fx_hbm, fx_v)
        pltpu.sync_copy(fy_hbm.at[pl.ds(base * 16, _TPW * 16)], fy_v)

        rsems = (rsem0, rsem1)

        def row_copies(k, slot):
            t = base + k
            h = lax.rem(t, 2)
            i = lax.rem(lax.div(t, 2), _OH)
            n = lax.div(t, 2 * _OH)
            y0 = lax.div(24 * i + 5, 14)
            s0 = (n * _H + y0) * 2 + h
            sem = rsems[slot]
            off = slot * 2 * _ROWLEN
            return (
                pltpu.make_async_copy(
                    img_hbm.at[s0], rows_v.at[pl.ds(off, _ROWLEN)], sem),
                pltpu.make_async_copy(
                    img_hbm.at[s0 + 2],
                    rows_v.at[pl.ds(off + _ROWLEN, _ROWLEN)], sem),
            )

        def start_rows(k, slot):
            c0, c1 = row_copies(k, slot)
            c0.start()
            c1.start()

        def wait_rows(k, slot):
            c0, c1 = row_copies(k, slot)
            c0.wait()
            c1.wait()

        start_rows(0, 0)

        def task(k, slot):
            @pl.when(k + 1 < _TPW)
            def _():
                start_rows(k + 1, slot ^ 1)
            wait_rows(k, slot)
            soff = slot * 2 * _ROWLEN
            fyv = fy_v[pl.ds(k * 16, 16)]
            jbase = lax.rem(base + k, 2) * _HALF

            @plsc.parallel_loop(0, _HALF, unroll=4)
            def jloop(j):
                jj = (jbase + j) * 16
                iav = xa_v[pl.ds(jj, 16)]
                fxv = fx_v[pl.ds(jj, 16)]
                obase = j * _C
                for c in range(0, _C, 16):
                    ia = iav + (soff + c)
                    a0 = plsc.load_gather(rows_v, [ia])
                    b0 = plsc.load_gather(rows_v, [ia + _C])
                    a1 = plsc.load_gather(rows_v, [ia + _ROWLEN])
                    b1 = plsc.load_gather(rows_v, [ia + (_ROWLEN + _C)])
                    t0 = a0 + fxv * (b0 - a0)
                    t1 = a1 + fxv * (b1 - a1)
                    out_v[pl.ds(obase + c, 16)] = t0 + fyv * (t1 - t0)

            pltpu.sync_copy(out_v, out_hbm.at[base + k])

        def pair(k2, carry):
            task(2 * k2, 0)
            task(2 * k2 + 1, 1)
            return carry

        lax.fori_loop(0, _TPW // 2, pair, 0)

    return run(img3, _XA, _FX, _FY)


def kernel(img):
    img3 = img.reshape(_N * _H * 2, _ROWLEN)
    out = _sc_resize(img3)
    return out.reshape(_N, _OH, _OW, _C)


# speedup vs baseline: 5.4437x; 3.5669x over previous
import functools

import numpy as np
import jax
import jax.numpy as jnp
from jax import lax
from jax.experimental import pallas as pl
from jax.experimental.pallas import tpu as pltpu
from jax.experimental.pallas import tpu_sc as plsc

_N, _H, _W, _C = 4, 384, 384, 96
_OH = _OW = 224
_CH = _C // 2
_NTASK = _N * _OH * 2
_NWORK = 32
_TPW = _NTASK // _NWORK
_NJC = _OW // 16


def _axis_tables():
    scale = np.float32(_W / _OW)
    i = np.arange(_OW, dtype=np.float32)
    src = (i + np.float32(0.5)) * scale - np.float32(0.5)
    src = np.maximum(src, np.float32(0.0))
    lo = np.floor(src).astype(np.int32)
    frac = (src - lo.astype(np.float32)).astype(np.float32)
    return lo, frac


def _tables():
    lo, frac = _axis_tables()
    x0_t = lo.copy()
    fx_t = frac.copy()
    t = np.arange(_NTASK, dtype=np.int32)
    i_out = (t // 2) % _OH
    fy_t = np.broadcast_to(frac[i_out][:, None], (_NTASK, 16)).astype(np.float32).reshape(-1).copy()
    return x0_t, fx_t, fy_t


_X0, _FX, _FY = _tables()


def _sc_resize(img3):
    mesh = plsc.VectorSubcoreMesh(core_axis_name="c", subcore_axis_name="s")

    @functools.partial(
        pl.kernel,
        out_type=jax.ShapeDtypeStruct((_NTASK, _CH, _OW), jnp.float32),
        mesh=mesh,
        scratch_types=[
            pltpu.VMEM((_OW,), jnp.int32),
            pltpu.VMEM((_OW,), jnp.float32),
            pltpu.VMEM((_TPW * 16,), jnp.float32),
            pltpu.VMEM((4 * _CH, _W), jnp.float32),
            pltpu.VMEM((_CH, _OW), jnp.float32),
            pltpu.SemaphoreType.DMA,
            pltpu.SemaphoreType.DMA,
        ],
        compiler_params=pltpu.CompilerParams(needs_layout_passes=False),
    )
    def run(img_hbm, x0_hbm, fx_hbm, fy_hbm, out_hbm,
            x0_v, fx_v, fy_v, rows_v, out_v, rsem0, rsem1):
        wid = lax.axis_index("s") * 2 + lax.axis_index("c")
        base = wid * _TPW
        pltpu.sync_copy(x0_hbm, x0_v)
        pltpu.sync_copy(fx_hbm, fx_v)
        pltpu.sync_copy(fy_hbm.at[pl.ds(base * 16, _TPW * 16)], fy_v)

        rsems = (rsem0, rsem1)
        zlane = lax.iota(jnp.int32, 16) * 0

        def row_copies(k, slot):
            t = base + k
            ch = lax.rem(t, 2)
            i = lax.rem(lax.div(t, 2), _OH)
            n = lax.div(t, 2 * _OH)
            y0 = lax.div(24 * i + 5, 14)
            s0 = (n * _H + y0) * 2 + ch
            sem = rsems[slot]
            off = slot * 2 * _CH
            return (
                pltpu.make_async_copy(
                    img_hbm.at[s0], rows_v.at[pl.ds(off, _CH)], sem),
                pltpu.make_async_copy(
                    img_hbm.at[s0 + 2],
                    rows_v.at[pl.ds(off + _CH, _CH)], sem),
            )

        def start_rows(k, slot):
            c0, c1 = row_copies(k, slot)
            c0.start()
            c1.start()

        def wait_rows(k, slot):
            c0, c1 = row_copies(k, slot)
            c0.wait()
            c1.wait()

        start_rows(0, 0)

        def task(k, slot):
            @pl.when(k + 1 < _TPW)
            def _():
                start_rows(k + 1, slot ^ 1)
            wait_rows(k, slot)
            rowbase = slot * 2 * _CH
            fyv = fy_v[pl.ds(k * 16, 16)]
            colvs = [x0_v[pl.ds(16 * jc, 16)] for jc in range(_NJC)]
            fxvs = [fx_v[pl.ds(16 * jc, 16)] for jc in range(_NJC)]

            @plsc.parallel_loop(0, _CH, unroll=2)
            def cloop(c):
                rowv0 = zlane + (c + rowbase)
                rowv1 = rowv0 + _CH
                for jc in range(_NJC):
                    colv = colvs[jc]
                    colb = colv + 1
                    a0 = plsc.load_gather(rows_v, [rowv0, colv])
                    b0 = plsc.load_gather(rows_v, [rowv0, colb])
                    a1 = plsc.load_gather(rows_v, [rowv1, colv])
                    b1 = plsc.load_gather(rows_v, [rowv1, colb])
                    fxv = fxvs[jc]
                    t0 = a0 + fxv * (b0 - a0)
                    t1 = a1 + fxv * (b1 - a1)
                    out_v[c, pl.ds(16 * jc, 16)] = t0 + fyv * (t1 - t0)

            pltpu.sync_copy(out_v, out_hbm.at[base + k])

        def pair(k2, carry):
            task(2 * k2, 0)
            task(2 * k2 + 1, 1)
            return carry

        lax.fori_loop(0, _TPW // 2, pair, 0)

    return run(img3, _X0, _FX, _FY)


def kernel(img):
    img_t = jnp.transpose(img, (0, 1, 3, 2))
    img3 = img_t.reshape(_N * _H, 2, _CH, _W).reshape(_N * _H * 2, _CH, _W)
    out = _sc_resize(img3)
    out_t = out.reshape(_N, _OH, 2, _CH, _OW).reshape(_N, _OH, _C, _OW)
    return jnp.transpose(out_t, (0, 1, 3, 2))
